# baseline (speedup 1.0000x reference)
import jax
import jax.numpy as jnp
from jax import lax
from jax.experimental import pallas as pl
from jax.experimental.pallas import tpu as pltpu


def _vq_tile_kernel(idx_ref, wt_ref, z_ref, zq_ref, cnt_ref, err_ref,
                    cnt_acc, err_acc):
    j = pl.program_id(1)
    nj = pl.num_programs(1)

    @pl.when(j == 0)
    def _():
        cnt_acc[...] = jnp.zeros_like(cnt_acc)
        err_acc[...] = jnp.zeros_like(err_acc)

    idx = idx_ref[...]
    k = wt_ref.shape[1]
    t = idx.shape[1]
    d = wt_ref.shape[0]

    row_iota = lax.broadcasted_iota(jnp.int32, (k, t), 0)
    onehot_t = (row_iota == idx).astype(jnp.bfloat16)

    zq_t = jnp.dot(wt_ref[...], onehot_t,
                   preferred_element_type=jnp.float32)
    zq_ref[...] = zq_t

    oh_f32 = onehot_t.astype(jnp.float32)
    cnt_acc[...] += jnp.sum(oh_f32.reshape(k, t // 128, 128), axis=1)

    diff = zq_t - z_ref[...]
    sq = diff * diff
    err_acc[...] += jnp.sum(sq.reshape(d // 8, 8, t), axis=0)

    @pl.when(j == nj - 1)
    def _():
        cnt_ref[...] = jnp.sum(cnt_acc[...], axis=1, keepdims=True)
        err_ref[...] = jnp.sum(err_acc[...]).reshape(1, 1)


def kernel(encoding_indices, z, weight, cluster_size_buf):
    b, d, h, w = z.shape
    hw = h * w
    n = b * hw
    k = weight.shape[0]
    beta = 0.25

    tile = 1024
    while hw % tile:
        tile //= 2
    n_tiles = hw // tile

    idx = encoding_indices.astype(jnp.int32).reshape(b, 1, hw)
    w_t = jnp.asarray(weight, jnp.float32).T.astype(jnp.bfloat16)
    z_flat = z.reshape(b, d, hw)

    grid = (b, n_tiles)
    idx_spec = pl.BlockSpec((None, 1, tile), lambda bi, j: (bi, 0, j))
    wt_spec = pl.BlockSpec(memory_space=pltpu.MemorySpace.VMEM)
    tok_spec = pl.BlockSpec((None, d, tile), lambda bi, j: (bi, 0, j))
    cnt_spec = pl.BlockSpec((None, k, 1), lambda bi, j: (bi, 0, 0))
    err_spec = pl.BlockSpec((None, 1, 1), lambda bi, j: (bi, 0, 0))

    cparams = pltpu.CompilerParams(
        dimension_semantics=("parallel", "arbitrary"),
        vmem_limit_bytes=64 << 20)

    zq_nc, cnt_part, err_part = pl.pallas_call(
        _vq_tile_kernel,
        out_shape=(
            jax.ShapeDtypeStruct((b, d, hw), jnp.float32),
            jax.ShapeDtypeStruct((b, k, 1), jnp.float32),
            jax.ShapeDtypeStruct((b, 1, 1), jnp.float32),
        ),
        grid_spec=pltpu.PrefetchScalarGridSpec(
            num_scalar_prefetch=0,
            grid=grid,
            in_specs=[idx_spec, wt_spec, tok_spec],
            out_specs=[tok_spec, cnt_spec, err_spec],
            scratch_shapes=[
                pltpu.VMEM((k, 128), jnp.float32),
                pltpu.VMEM((8, tile), jnp.float32),
            ],
        ),
        compiler_params=cparams,
    )(idx, w_t, z_flat)

    z_q = zq_nc.reshape(b, d, h, w)
    loss = beta * jnp.sum(err_part) / jnp.float32(n * d)
    counts = jnp.sum(cnt_part[:, :, 0], axis=0)
    new_cluster_size = counts + 0.0 * cluster_size_buf

    return z_q, loss, encoding_indices, new_cluster_size


# speedup vs baseline: 1.5820x; 1.5820x over previous
import jax
import jax.numpy as jnp
from jax import lax
from jax.experimental import pallas as pl
from jax.experimental.pallas import tpu as pltpu


def _vq_batch_kernel(idx_ref, wt_ref, z_ref, zq_ref, cnt_ref, err_ref):
    idx = idx_ref[...]
    d, k = wt_ref.shape
    t = idx.shape[1]

    row_iota = lax.broadcasted_iota(jnp.int32, (k, t), 0)
    mask = row_iota == idx
    onehot_bf = mask.astype(jnp.bfloat16)

    zq = jnp.dot(wt_ref[...], onehot_bf,
                 preferred_element_type=jnp.float32)
    zq_ref[...] = zq

    cnt_ref[...] = jnp.sum(mask.astype(jnp.float32), axis=1, keepdims=True)

    diff = zq - z_ref[...]
    sq = diff * diff
    err_ref[...] = jnp.sum(jnp.sum(sq.reshape(d // 8, 8, t), axis=0)
                           ).reshape(1, 1)


def kernel(encoding_indices, z, weight, cluster_size_buf):
    b, d, h, w = z.shape
    hw = h * w
    n = b * hw
    k = weight.shape[0]
    beta = 0.25

    idx = encoding_indices.astype(jnp.int32).reshape(b, 1, hw)
    w_t = jnp.asarray(weight, jnp.float32).T.astype(jnp.bfloat16)
    z_flat = z.reshape(b, d, hw)

    grid = (b,)
    idx_spec = pl.BlockSpec((None, 1, hw), lambda bi: (bi, 0, 0))
    wt_spec = pl.BlockSpec(memory_space=pltpu.MemorySpace.VMEM)
    tok_spec = pl.BlockSpec((None, d, hw), lambda bi: (bi, 0, 0))
    cnt_spec = pl.BlockSpec((None, k, 1), lambda bi: (bi, 0, 0))
    err_spec = pl.BlockSpec((None, 1, 1), lambda bi: (bi, 0, 0))

    cparams = pltpu.CompilerParams(
        dimension_semantics=("parallel",),
        vmem_limit_bytes=64 << 20)

    zq_nc, cnt_part, err_part = pl.pallas_call(
        _vq_batch_kernel,
        out_shape=(
            jax.ShapeDtypeStruct((b, d, hw), jnp.float32),
            jax.ShapeDtypeStruct((b, k, 1), jnp.float32),
            jax.ShapeDtypeStruct((b, 1, 1), jnp.float32),
        ),
        grid_spec=pltpu.PrefetchScalarGridSpec(
            num_scalar_prefetch=0,
            grid=grid,
            in_specs=[idx_spec, wt_spec, tok_spec],
            out_specs=[tok_spec, cnt_spec, err_spec],
        ),
        compiler_params=cparams,
    )(idx, w_t, z_flat)

    z_q = zq_nc.reshape(b, d, h, w)
    loss = beta * jnp.sum(err_part) / jnp.float32(n * d)
    counts = jnp.sum(cnt_part[:, :, 0], axis=0)
    new_cluster_size = counts + 0.0 * cluster_size_buf

    return z_q, loss, encoding_indices, new_cluster_size
